# Initial kernel scaffold; baseline (speedup 1.0000x reference)
import jax
import jax.numpy as jnp
from jax.experimental import pallas as pl


def kernel(graph_x, graph_edge_index, connected_trigger_node_index, target_node_index, W1, b1, W2, b2, fgW1, fgb1, fgW2, fgb2, egW1, egb1, egW2, egb2):
    raise NotImplementedError("write your pallas kernel here")


# speedup vs baseline: 25.0635x; 25.0635x over previous
import functools

import numpy as np
import jax
import jax.numpy as jnp
from jax import lax
from jax.experimental import pallas as pl
from jax.experimental.pallas import tpu as pltpu
from jax.experimental.pallas import tpu_sc as plsc

_N = 10000
_E = 320000
_D = 128
_HID = 256
_T = 16

_NC, _NS, _L = 2, 16, 16
_NW = _NC * _NS
_NPAD = 10240
_RPS = _NPAD // _NS
_EPW = _E // _NW
_K = 80
_NCHUNK = _EPW // _K

_mesh = plsc.VectorSubcoreMesh(core_axis_name="c", subcore_axis_name="s")

_OFF = np.array([(i, j) for i in range(_T) for j in range(_T) if i != j],
                dtype=np.int32)
_OFF_FLAT = _OFF[:, 0] * _T + _OFF[:, 1]


@functools.partial(
    pl.kernel,
    out_type=jax.ShapeDtypeStruct((_NC, _NPAD), jnp.float32),
    mesh=_mesh,
    scratch_types=[
        pltpu.VMEM((_K,), jnp.int32),
        pltpu.VMEM((_K,), jnp.float32),
        pltpu.VMEM((_RPS,), jnp.float32),
        pltpu.VMEM_SHARED((_NPAD,), jnp.float32),
    ],
)
def _sc_deg(dst_hbm, zeros_hbm, out_hbm, idx_v, ones_v, zb_v, hist_sp):
    cid = lax.axis_index("c")
    sid = lax.axis_index("s")
    pltpu.sync_copy(zeros_hbm, zb_v)
    pltpu.sync_copy(zb_v, hist_sp.at[pl.ds(sid * _RPS, _RPS)])
    for j in range(_K // _L):
        ones_v[pl.ds(j * _L, _L)] = jnp.ones((_L,), jnp.float32)
    plsc.subcore_barrier()
    w = cid * _NS + sid

    def step(i, carry):
        base = w * _EPW + i * _K
        pltpu.sync_copy(dst_hbm.at[pl.ds(base, _K)], idx_v)
        pltpu.sync_copy(ones_v, hist_sp.at[idx_v], add=True)
        return carry

    lax.fori_loop(0, _NCHUNK, step, 0)
    plsc.subcore_barrier()
    pltpu.sync_copy(hist_sp.at[pl.ds(sid * _RPS, _RPS)],
                    out_hbm.at[cid, pl.ds(sid * _RPS, _RPS)])


@functools.partial(
    pl.kernel,
    out_type=(
        jax.ShapeDtypeStruct((_NC, _NPAD, _D), jnp.float32),
        jax.ShapeDtypeStruct((_NC, _NPAD), jnp.float32),
    ),
    mesh=_mesh,
    scratch_types=[
        pltpu.VMEM_SHARED((_NPAD,), jnp.float32),
        pltpu.VMEM((_K,), jnp.int32),
        pltpu.VMEM((_K,), jnp.int32),
        pltpu.VMEM((_K,), jnp.float32),
        pltpu.VMEM((_K, _D), jnp.float32),
        pltpu.VMEM((_RPS,), jnp.float32),
        pltpu.VMEM_SHARED((_NPAD, _D), jnp.float32),
        pltpu.VMEM_SHARED((_NPAD,), jnp.float32),
        pltpu.SemaphoreType.DMA,
    ],
)
def _sc_scatter(src_hbm, dst_hbm, xs_hbm, dinv_hbm, zrows_hbm, zeros_hbm,
                zout_hbm, uout_hbm,
                dinv_sp, sidx_v, didx_v, val_v, rows_v, zb_v, z_sp, u_sp, sem):
    cid = lax.axis_index("c")
    sid = lax.axis_index("s")
    r0 = sid * _RPS
    pltpu.sync_copy(dinv_hbm.at[pl.ds(r0, _RPS)], dinv_sp.at[pl.ds(r0, _RPS)])
    pltpu.sync_copy(zrows_hbm, rows_v)
    pltpu.sync_copy(zeros_hbm, zb_v)
    for r in range(_RPS // _K):
        pltpu.sync_copy(rows_v, z_sp.at[pl.ds(r0 + r * _K, _K), :])
    pltpu.sync_copy(zb_v, u_sp.at[pl.ds(r0, _RPS)])
    plsc.subcore_barrier()
    w = cid * _NS + sid

    def step(i, carry):
        base = w * _EPW + i * _K
        pltpu.sync_copy(src_hbm.at[pl.ds(base, _K)], sidx_v)
        pltpu.sync_copy(dst_hbm.at[pl.ds(base, _K)], didx_v)
        pltpu.sync_copy(dinv_sp.at[didx_v], val_v)
        pltpu.async_copy(xs_hbm.at[sidx_v], rows_v, sem).wait()
        pltpu.sync_copy(val_v, u_sp.at[sidx_v], add=True)
        pltpu.sync_copy(rows_v, z_sp.at[didx_v], add=True)
        return carry

    lax.fori_loop(0, _NCHUNK, step, 0)
    plsc.subcore_barrier()
    pltpu.sync_copy(z_sp.at[pl.ds(r0, _RPS), :],
                    zout_hbm.at[cid, pl.ds(r0, _RPS), :])
    pltpu.sync_copy(u_sp.at[pl.ds(r0, _RPS)],
                    uout_hbm.at[cid, pl.ds(r0, _RPS)])


_BA = 2048


def _prep_body(degp_ref, x_ref, dinv_ref, xs_ref):
    deg = degp_ref[:, 0:1] + degp_ref[:, 1:2] + 1.0
    dv = lax.rsqrt(deg)
    dinv_ref[...] = dv
    xs_ref[...] = x_ref[...] * dv


def _tc_prep(degp_t, x_pad):
    return pl.pallas_call(
        _prep_body,
        grid=(_NPAD // _BA,),
        in_specs=[
            pl.BlockSpec((_BA, 2), lambda i: (i, 0)),
            pl.BlockSpec((_BA, _D), lambda i: (i, 0)),
        ],
        out_specs=[
            pl.BlockSpec((_BA, 1), lambda i: (i, 0)),
            pl.BlockSpec((_BA, _D), lambda i: (i, 0)),
        ],
        out_shape=[
            jax.ShapeDtypeStruct((_NPAD, 1), jnp.float32),
            jax.ShapeDtypeStruct((_NPAD, _D), jnp.float32),
        ],
    )(degp_t, x_pad)


_BB = 1024


def _red_body(zp_ref, up_ref, dinv_ref, x_ref, w1_ref, b1_ref, s_ref):
    i = pl.program_id(0)
    dv = dinv_ref[...]
    z = dv * (zp_ref[0] + zp_ref[1]) + dv * dv * x_ref[...]
    y = jnp.dot(z, w1_ref[...], preferred_element_type=jnp.float32) + b1_ref[...]
    y = jnp.where(y >= 0.0, y, 0.01 * y)
    u = up_ref[:, 0:1] + up_ref[:, 1:2]
    wv = dv * (u + dv)
    rows = i * _BB + lax.broadcasted_iota(jnp.int32, (_BB, 1), 0)
    wv = jnp.where(rows < _N, wv, 0.0)
    part = jnp.sum(wv * y, axis=0, keepdims=True)

    @pl.when(i == 0)
    def _():
        s_ref[...] = part

    @pl.when(i > 0)
    def _():
        s_ref[...] += part


def _tc_reduce(zp, up_t, dinv2, x_pad, W1, b1r):
    return pl.pallas_call(
        _red_body,
        grid=(_NPAD // _BB,),
        in_specs=[
            pl.BlockSpec((_NC, _BB, _D), lambda i: (0, i, 0)),
            pl.BlockSpec((_BB, 2), lambda i: (i, 0)),
            pl.BlockSpec((_BB, 1), lambda i: (i, 0)),
            pl.BlockSpec((_BB, _D), lambda i: (i, 0)),
            pl.BlockSpec((_D, _HID), lambda i: (0, 0)),
            pl.BlockSpec((1, _HID), lambda i: (0, 0)),
        ],
        out_specs=pl.BlockSpec((1, _HID), lambda i: (0, 0)),
        out_shape=jax.ShapeDtypeStruct((1, _HID), jnp.float32),
    )(zp, up_t, dinv2, x_pad, W1, b1r)


def _head_body(ci_ref, s_ref, w2_ref, b2_ref, fg1_ref, fgb1_ref, fg2_ref,
               fgb2_ref, eg1_ref, egb1_ref, eg2_ref, egb2_ref, tf_ref, p_ref):
    g = jnp.dot(s_ref[...], w2_ref[...],
                preferred_element_type=jnp.float32) * (1.0 / _N) + b2_ref[...]
    h1 = jnp.maximum(jnp.dot(g, fg1_ref[...],
                             preferred_element_type=jnp.float32) + fgb1_ref[...], 0.0)
    tf_ref[...] = jnp.dot(h1, fg2_ref[...],
                          preferred_element_type=jnp.float32) + fgb2_ref[...]
    h2 = jnp.maximum(jnp.dot(g, eg1_ref[...],
                             preferred_element_type=jnp.float32) + egb1_ref[...], 0.0)
    ew = jnp.dot(h2, eg2_ref[...],
                 preferred_element_type=jnp.float32) + egb2_ref[...]
    p = jax.nn.sigmoid(ew)
    k = lax.broadcasted_iota(jnp.int32, (1, _T * _T), 1)
    ii = k // _T
    jj = k - ii * _T
    ci = ci_ref[0]
    p = jnp.where((ii == ci) | (jj == ci), 1.0, p)
    p = jnp.where(ii == jj, 0.0, p)
    p_ref[...] = p


def _tc_head(ci_arr, s, W2, b2r, fgW1, fgb1r, fgW2, fgb2r, egW1, egb1r, egW2, egb2r):
    return pl.pallas_call(
        _head_body,
        in_specs=[pl.BlockSpec(memory_space=pltpu.SMEM)] + [pl.BlockSpec()] * 11,
        out_specs=[pl.BlockSpec(), pl.BlockSpec()],
        out_shape=[
            jax.ShapeDtypeStruct((1, _T * _D), jnp.float32),
            jax.ShapeDtypeStruct((1, _T * _T), jnp.float32),
        ],
    )(ci_arr, s, W2, b2r, fgW1, fgb1r, fgW2, fgb2r, egW1, egb1r, egW2, egb2r)


def kernel(graph_x, graph_edge_index, connected_trigger_node_index,
           target_node_index, W1, b1, W2, b2, fgW1, fgb1, fgW2, fgb2,
           egW1, egb1, egW2, egb2):
    ci = jnp.asarray(connected_trigger_node_index, dtype=jnp.int32)
    ti = jnp.asarray(target_node_index, dtype=jnp.int32)
    x_pad = jnp.pad(graph_x, ((0, _NPAD - _N), (0, 0)))
    zeros1d = jnp.zeros((_RPS,), jnp.float32)
    zrows = jnp.zeros((_K, _D), jnp.float32)

    src = graph_edge_index[0]
    dst = graph_edge_index[1]
    degp = _sc_deg(dst, zeros1d)
    dinv2, xs = _tc_prep(degp.T, x_pad)
    zp, up = _sc_scatter(src, dst, xs, dinv2.reshape(_NPAD),
                         zrows, zeros1d)
    s = _tc_reduce(zp, up.T, dinv2, x_pad, W1, b1.reshape(1, _HID))
    tf, p = _tc_head(ci.reshape(1), s, W2, b2.reshape(1, _HID),
                     fgW1, fgb1.reshape(1, _HID), fgW2, fgb2.reshape(1, _T * _D),
                     egW1, egb1.reshape(1, _HID), egW2, egb2.reshape(1, _T * _T))

    trigger_features = tf.reshape(_T, _D)
    trigger_edge_weight = p.reshape(_T * _T)[jnp.asarray(_OFF_FLAT)]
    rows = jnp.asarray(_OFF[:, 0], jnp.int32)
    cols = jnp.asarray(_OFF[:, 1], jnp.int32)
    trigger_edge_index = jnp.stack([rows + _N, cols + _N]).astype(jnp.int32)
    tte = jnp.stack([jnp.stack([ci + _N, ti]),
                     jnp.stack([ti, ci + _N])]).astype(jnp.int32)
    combined_x = jnp.concatenate([graph_x, trigger_features], axis=0)
    combined_edge_index = jnp.concatenate(
        [graph_edge_index, tte, trigger_edge_index], axis=1)
    return (combined_x, combined_edge_index, trigger_edge_weight)


# speedup vs baseline: 44.2216x; 1.7644x over previous
import functools

import numpy as np
import jax
import jax.numpy as jnp
from jax import lax
from jax.experimental import pallas as pl
from jax.experimental.pallas import tpu as pltpu
from jax.experimental.pallas import tpu_sc as plsc

_N = 10000
_E = 320000
_D = 128
_HID = 256
_T = 16

_NC, _NS, _L = 2, 16, 16
_NW = _NC * _NS
_NPAD = 10240
_RPS = _NPAD // _NS
_EPW = _E // _NW
_K = 80
_NCHUNK = _EPW // _K

_mesh = plsc.VectorSubcoreMesh(core_axis_name="c", subcore_axis_name="s")

_OFF = np.array([(i, j) for i in range(_T) for j in range(_T) if i != j],
                dtype=np.int32)
_OFF_FLAT = _OFF[:, 0] * _T + _OFF[:, 1]


@functools.partial(
    pl.kernel,
    out_type=jax.ShapeDtypeStruct((_NC, _NPAD), jnp.float32),
    mesh=_mesh,
    scratch_types=[
        pltpu.VMEM((_K,), jnp.int32),
        pltpu.VMEM((_K,), jnp.int32),
        pltpu.VMEM((_K,), jnp.float32),
        pltpu.VMEM((_RPS,), jnp.float32),
        pltpu.VMEM_SHARED((_NPAD,), jnp.float32),
        pltpu.SemaphoreType.DMA,
        pltpu.SemaphoreType.DMA,
    ],
)
def _sc_deg(dst_hbm, zeros_hbm, out_hbm, didx0, didx1, ones_v, zb_v, hist_sp,
            isem0, isem1):
    cid = lax.axis_index("c")
    sid = lax.axis_index("s")
    w = cid * _NS + sid
    e0 = w * _EPW
    pltpu.sync_copy(zeros_hbm, zb_v)
    pltpu.sync_copy(zb_v, hist_sp.at[pl.ds(sid * _RPS, _RPS)])
    for j in range(_K // _L):
        ones_v[pl.ds(j * _L, _L)] = jnp.ones((_L,), jnp.float32)
    plsc.subcore_barrier()
    pltpu.async_copy(dst_hbm.at[pl.ds(e0, _K)], didx0, isem0)
    pltpu.async_copy(dst_hbm.at[pl.ds(e0 + _K, _K)], didx1, isem1)

    def process(c, dic, isc, prefetch_i):
        pltpu.make_async_copy(dst_hbm.at[pl.ds(0, _K)], dic, isc).wait()
        pltpu.sync_copy(ones_v, hist_sp.at[dic], add=True)
        if prefetch_i:
            @pl.when(c + 2 < _NCHUNK)
            def _():
                pltpu.async_copy(dst_hbm.at[pl.ds(e0 + (c + 2) * _K, _K)],
                                 dic, isc)

    def step(p, carry):
        c = 2 * p
        process(c, didx0, isem0, True)
        process(c + 1, didx1, isem1, True)
        return carry

    lax.fori_loop(0, _NCHUNK // 2, step, 0)
    process(_NCHUNK - 1, didx0, isem0, False)
    plsc.subcore_barrier()
    pltpu.sync_copy(hist_sp.at[pl.ds(sid * _RPS, _RPS)],
                    out_hbm.at[cid, pl.ds(sid * _RPS, _RPS)])


@functools.partial(
    pl.kernel,
    out_type=(
        jax.ShapeDtypeStruct((_NC, _NPAD, _D), jnp.float32),
        jax.ShapeDtypeStruct((_NC, _NPAD), jnp.float32),
    ),
    mesh=_mesh,
    scratch_types=[
        pltpu.VMEM((_K,), jnp.int32),
        pltpu.VMEM((_K,), jnp.int32),
        pltpu.VMEM((_K,), jnp.int32),
        pltpu.VMEM((_K,), jnp.int32),
        pltpu.VMEM((_K,), jnp.float32),
        pltpu.VMEM((_K, _D), jnp.float32),
        pltpu.VMEM((_K, _D), jnp.float32),
        pltpu.VMEM((_RPS,), jnp.float32),
        pltpu.VMEM_SHARED((_NPAD,), jnp.float32),
        pltpu.VMEM_SHARED((_NPAD, _D), jnp.float32),
        pltpu.VMEM_SHARED((_NPAD,), jnp.float32),
        pltpu.SemaphoreType.DMA,
        pltpu.SemaphoreType.DMA,
        pltpu.SemaphoreType.DMA,
        pltpu.SemaphoreType.DMA,
    ],
)
def _sc_scatter(src_hbm, dst_hbm, xs_hbm, dinv_hbm, zrows_hbm, zeros_hbm,
                zout_hbm, uout_hbm,
                sidx0, didx0, sidx1, didx1, val_v, rows0, rows1, zb_v,
                dinv_sp, z_sp, u_sp, isem0, isem1, gsem0, gsem1):
    cid = lax.axis_index("c")
    sid = lax.axis_index("s")
    w = cid * _NS + sid
    e0 = w * _EPW
    r0 = sid * _RPS
    pltpu.sync_copy(dinv_hbm.at[pl.ds(r0, _RPS)], dinv_sp.at[pl.ds(r0, _RPS)])
    pltpu.sync_copy(zrows_hbm, rows0)
    pltpu.sync_copy(zeros_hbm, zb_v)
    for r in range(_RPS // _K):
        pltpu.sync_copy(rows0, z_sp.at[pl.ds(r0 + r * _K, _K), :])
    pltpu.sync_copy(zb_v, u_sp.at[pl.ds(r0, _RPS)])
    plsc.subcore_barrier()

    def fetch_idx(c, sic, dic, isc):
        pltpu.async_copy(src_hbm.at[pl.ds(e0 + c * _K, _K)], sic, isc)
        pltpu.async_copy(dst_hbm.at[pl.ds(e0 + c * _K, _K)], dic, isc)

    def wait_idx(sic, dic, isc):
        pltpu.make_async_copy(src_hbm.at[pl.ds(0, _K)], sic, isc).wait()
        pltpu.make_async_copy(src_hbm.at[pl.ds(0, _K)], dic, isc).wait()

    fetch_idx(0, sidx0, didx0, isem0)
    wait_idx(sidx0, didx0, isem0)
    pltpu.async_copy(xs_hbm.at[sidx0], rows0, gsem0)
    fetch_idx(1, sidx1, didx1, isem1)

    def process(c, sic, dic, rc, gc, isc, sin, din, rn, gn, isn,
                pre_gather, pre_idx):
        if pre_gather:
            wait_idx(sin, din, isn)
            pltpu.async_copy(xs_hbm.at[sin], rn, gn)
        pltpu.sync_copy(dinv_sp.at[dic], val_v)
        pltpu.sync_copy(val_v, u_sp.at[sic], add=True)
        pltpu.make_async_copy(xs_hbm.at[sic], rc, gc).wait()
        pltpu.sync_copy(rc, z_sp.at[dic], add=True)
        if pre_idx:
            @pl.when(c + 2 < _NCHUNK)
            def _():
                fetch_idx(c + 2, sic, dic, isc)

    b0 = (sidx0, didx0, rows0, gsem0, isem0)
    b1 = (sidx1, didx1, rows1, gsem1, isem1)

    def step(p, carry):
        c = 2 * p
        process(c, *b0, *b1, True, True)
        process(c + 1, *b1, *b0, True, True)
        return carry

    lax.fori_loop(0, _NCHUNK // 2, step, 0)
    process(_NCHUNK - 1, *b0, *b1, False, False)
    plsc.subcore_barrier()
    pltpu.sync_copy(z_sp.at[pl.ds(r0, _RPS), :],
                    zout_hbm.at[cid, pl.ds(r0, _RPS), :])
    pltpu.sync_copy(u_sp.at[pl.ds(r0, _RPS)],
                    uout_hbm.at[cid, pl.ds(r0, _RPS)])


_BA = 2048


def _prep_body(degp_ref, x_ref, dinv_ref, xs_ref):
    deg = degp_ref[:, 0:1] + degp_ref[:, 1:2] + 1.0
    dv = lax.rsqrt(deg)
    dinv_ref[...] = dv
    xs_ref[...] = x_ref[...] * dv


def _tc_prep(degp_t, x_pad):
    return pl.pallas_call(
        _prep_body,
        grid=(_NPAD // _BA,),
        in_specs=[
            pl.BlockSpec((_BA, 2), lambda i: (i, 0)),
            pl.BlockSpec((_BA, _D), lambda i: (i, 0)),
        ],
        out_specs=[
            pl.BlockSpec((_BA, 1), lambda i: (i, 0)),
            pl.BlockSpec((_BA, _D), lambda i: (i, 0)),
        ],
        out_shape=[
            jax.ShapeDtypeStruct((_NPAD, 1), jnp.float32),
            jax.ShapeDtypeStruct((_NPAD, _D), jnp.float32),
        ],
    )(degp_t, x_pad)


_BB = 1024


def _red_body(zp_ref, up_ref, dinv_ref, x_ref, w1_ref, b1_ref, s_ref):
    i = pl.program_id(0)
    dv = dinv_ref[...]
    z = dv * (zp_ref[0] + zp_ref[1]) + dv * dv * x_ref[...]
    y = jnp.dot(z, w1_ref[...], preferred_element_type=jnp.float32) + b1_ref[...]
    y = jnp.where(y >= 0.0, y, 0.01 * y)
    u = up_ref[:, 0:1] + up_ref[:, 1:2]
    wv = dv * (u + dv)
    rows = i * _BB + lax.broadcasted_iota(jnp.int32, (_BB, 1), 0)
    wv = jnp.where(rows < _N, wv, 0.0)
    part = jnp.sum(wv * y, axis=0, keepdims=True)

    @pl.when(i == 0)
    def _():
        s_ref[...] = part

    @pl.when(i > 0)
    def _():
        s_ref[...] += part


def _tc_reduce(zp, up_t, dinv2, x_pad, W1, b1r):
    return pl.pallas_call(
        _red_body,
        grid=(_NPAD // _BB,),
        in_specs=[
            pl.BlockSpec((_NC, _BB, _D), lambda i: (0, i, 0)),
            pl.BlockSpec((_BB, 2), lambda i: (i, 0)),
            pl.BlockSpec((_BB, 1), lambda i: (i, 0)),
            pl.BlockSpec((_BB, _D), lambda i: (i, 0)),
            pl.BlockSpec((_D, _HID), lambda i: (0, 0)),
            pl.BlockSpec((1, _HID), lambda i: (0, 0)),
        ],
        out_specs=pl.BlockSpec((1, _HID), lambda i: (0, 0)),
        out_shape=jax.ShapeDtypeStruct((1, _HID), jnp.float32),
    )(zp, up_t, dinv2, x_pad, W1, b1r)


def _head_body(ci_ref, s_ref, w2_ref, b2_ref, fg1_ref, fgb1_ref, fg2_ref,
               fgb2_ref, eg1_ref, egb1_ref, eg2_ref, egb2_ref, tf_ref, p_ref):
    g = jnp.dot(s_ref[...], w2_ref[...],
                preferred_element_type=jnp.float32) * (1.0 / _N) + b2_ref[...]
    h1 = jnp.maximum(jnp.dot(g, fg1_ref[...],
                             preferred_element_type=jnp.float32) + fgb1_ref[...], 0.0)
    tf_ref[...] = jnp.dot(h1, fg2_ref[...],
                          preferred_element_type=jnp.float32) + fgb2_ref[...]
    h2 = jnp.maximum(jnp.dot(g, eg1_ref[...],
                             preferred_element_type=jnp.float32) + egb1_ref[...], 0.0)
    ew = jnp.dot(h2, eg2_ref[...],
                 preferred_element_type=jnp.float32) + egb2_ref[...]
    p = jax.nn.sigmoid(ew)
    k = lax.broadcasted_iota(jnp.int32, (1, _T * _T), 1)
    ii = k // _T
    jj = k - ii * _T
    ci = ci_ref[0]
    p = jnp.where((ii == ci) | (jj == ci), 1.0, p)
    p = jnp.where(ii == jj, 0.0, p)
    p_ref[...] = p


def _tc_head(ci_arr, s, W2, b2r, fgW1, fgb1r, fgW2, fgb2r, egW1, egb1r, egW2, egb2r):
    return pl.pallas_call(
        _head_body,
        in_specs=[pl.BlockSpec(memory_space=pltpu.SMEM)] + [pl.BlockSpec()] * 11,
        out_specs=[pl.BlockSpec(), pl.BlockSpec()],
        out_shape=[
            jax.ShapeDtypeStruct((1, _T * _D), jnp.float32),
            jax.ShapeDtypeStruct((1, _T * _T), jnp.float32),
        ],
    )(ci_arr, s, W2, b2r, fgW1, fgb1r, fgW2, fgb2r, egW1, egb1r, egW2, egb2r)


def kernel(graph_x, graph_edge_index, connected_trigger_node_index,
           target_node_index, W1, b1, W2, b2, fgW1, fgb1, fgW2, fgb2,
           egW1, egb1, egW2, egb2):
    ci = jnp.asarray(connected_trigger_node_index, dtype=jnp.int32)
    ti = jnp.asarray(target_node_index, dtype=jnp.int32)
    x_pad = jnp.pad(graph_x, ((0, _NPAD - _N), (0, 0)))
    zeros1d = jnp.zeros((_RPS,), jnp.float32)
    zrows = jnp.zeros((_K, _D), jnp.float32)

    src = graph_edge_index[0]
    dst = graph_edge_index[1]
    degp = _sc_deg(dst, zeros1d)
    dinv2, xs = _tc_prep(degp.T, x_pad)
    zp, up = _sc_scatter(src, dst, xs, dinv2.reshape(_NPAD),
                         zrows, zeros1d)
    s = _tc_reduce(zp, up.T, dinv2, x_pad, W1, b1.reshape(1, _HID))
    tf, p = _tc_head(ci.reshape(1), s, W2, b2.reshape(1, _HID),
                     fgW1, fgb1.reshape(1, _HID), fgW2, fgb2.reshape(1, _T * _D),
                     egW1, egb1.reshape(1, _HID), egW2, egb2.reshape(1, _T * _T))

    trigger_features = tf.reshape(_T, _D)
    trigger_edge_weight = p.reshape(_T * _T)[jnp.asarray(_OFF_FLAT)]
    rows = jnp.asarray(_OFF[:, 0], jnp.int32)
    cols = jnp.asarray(_OFF[:, 1], jnp.int32)
    trigger_edge_index = jnp.stack([rows + _N, cols + _N]).astype(jnp.int32)
    tte = jnp.stack([jnp.stack([ci + _N, ti]),
                     jnp.stack([ti, ci + _N])]).astype(jnp.int32)
    combined_x = jnp.concatenate([graph_x, trigger_features], axis=0)
    combined_edge_index = jnp.concatenate(
        [graph_edge_index, tte, trigger_edge_index], axis=1)
    return (combined_x, combined_edge_index, trigger_edge_weight)


# speedup vs baseline: 46.9141x; 1.0609x over previous
import functools

import numpy as np
import jax
import jax.numpy as jnp
from jax import lax
from jax.experimental import pallas as pl
from jax.experimental.pallas import tpu as pltpu
from jax.experimental.pallas import tpu_sc as plsc

_N = 10000
_E = 320000
_D = 128
_HID = 256
_T = 16

_NC, _NS, _L = 2, 16, 16
_NW = _NC * _NS
_NPAD = 10240
_RPS = _NPAD // _NS
_EPW = _E // _NW
_K = 80
_NCHUNK = _EPW // _K

_mesh = plsc.VectorSubcoreMesh(core_axis_name="c", subcore_axis_name="s")

_OFF = np.array([(i, j) for i in range(_T) for j in range(_T) if i != j],
                dtype=np.int32)
_OFF_FLAT = _OFF[:, 0] * _T + _OFF[:, 1]


@functools.partial(
    pl.kernel,
    out_type=jax.ShapeDtypeStruct((_NC, _NPAD), jnp.float32),
    mesh=_mesh,
    scratch_types=[
        pltpu.VMEM((_K,), jnp.int32),
        pltpu.VMEM((_K,), jnp.int32),
        pltpu.VMEM((_K,), jnp.int32),
        pltpu.VMEM((_K,), jnp.float32),
        pltpu.VMEM((_RPS,), jnp.float32),
        pltpu.VMEM_SHARED((_NPAD,), jnp.float32),
        pltpu.SemaphoreType.DMA,
        pltpu.SemaphoreType.DMA,
        pltpu.SemaphoreType.DMA,
        pltpu.SemaphoreType.DMA,
        pltpu.SemaphoreType.DMA,
        pltpu.SemaphoreType.DMA,
    ],
)
def _sc_deg(dst_hbm, zeros_hbm, out_hbm, didx0, didx1, didx2, ones_v, zb_v,
            hist_sp, isem0, isem1, isem2, ssem0, ssem1, ssem2):
    cid = lax.axis_index("c")
    sid = lax.axis_index("s")
    w = cid * _NS + sid
    e0 = w * _EPW
    pltpu.sync_copy(zeros_hbm, zb_v)
    pltpu.sync_copy(zb_v, hist_sp.at[pl.ds(sid * _RPS, _RPS)])
    for j in range(_K // _L):
        ones_v[pl.ds(j * _L, _L)] = jnp.ones((_L,), jnp.float32)
    plsc.subcore_barrier()
    bufs = ((didx0, isem0, ssem0), (didx1, isem1, ssem1), (didx2, isem2, ssem2))
    pltpu.async_copy(dst_hbm.at[pl.ds(e0, _K)], didx0, isem0)
    pltpu.async_copy(dst_hbm.at[pl.ds(e0 + _K, _K)], didx1, isem1)

    def process(c, cur, prv):
        dic, isc, ssc = cur
        dip, isp, ssp = prv
        pltpu.make_async_copy(dst_hbm.at[pl.ds(0, _K)], dic, isc).wait()
        pltpu.async_copy(ones_v, hist_sp.at[dic], ssc, add=True)

        @pl.when(c >= 1)
        def _():
            pltpu.make_async_copy(ones_v, hist_sp.at[dip], ssp).wait()

        @pl.when(c + 2 < _NCHUNK)
        def _():
            pltpu.async_copy(dst_hbm.at[pl.ds(e0 + (c + 2) * _K, _K)],
                             dip, isp)

    def step(p, carry):
        c = 3 * p
        process(c, bufs[0], bufs[2])
        process(c + 1, bufs[1], bufs[0])
        process(c + 2, bufs[2], bufs[1])
        return carry

    lax.fori_loop(0, _NCHUNK // 3, step, 0)
    process(_NCHUNK - 2, bufs[0], bufs[2])
    process(_NCHUNK - 1, bufs[1], bufs[0])
    pltpu.make_async_copy(ones_v, hist_sp.at[didx1], ssem1).wait()
    plsc.subcore_barrier()
    pltpu.sync_copy(hist_sp.at[pl.ds(sid * _RPS, _RPS)],
                    out_hbm.at[cid, pl.ds(sid * _RPS, _RPS)])


@functools.partial(
    pl.kernel,
    out_type=(
        jax.ShapeDtypeStruct((_NC, _NPAD, _D), jnp.float32),
        jax.ShapeDtypeStruct((_NC, _NPAD), jnp.float32),
    ),
    mesh=_mesh,
    scratch_types=[
        pltpu.VMEM((_K,), jnp.int32),
        pltpu.VMEM((_K,), jnp.int32),
        pltpu.VMEM((_K,), jnp.int32),
        pltpu.VMEM((_K,), jnp.int32),
        pltpu.VMEM((_K,), jnp.float32),
        pltpu.VMEM((_K,), jnp.float32),
        pltpu.VMEM((_K, _D), jnp.float32),
        pltpu.VMEM((_K, _D), jnp.float32),
        pltpu.VMEM((_RPS,), jnp.float32),
        pltpu.VMEM_SHARED((_NPAD,), jnp.float32),
        pltpu.VMEM_SHARED((_NPAD, _D), jnp.float32),
        pltpu.VMEM_SHARED((_NPAD,), jnp.float32),
        pltpu.SemaphoreType.DMA,
        pltpu.SemaphoreType.DMA,
        pltpu.SemaphoreType.DMA,
        pltpu.SemaphoreType.DMA,
        pltpu.SemaphoreType.DMA,
        pltpu.SemaphoreType.DMA,
        pltpu.SemaphoreType.DMA,
        pltpu.SemaphoreType.DMA,
    ],
)
def _sc_scatter(src_hbm, dst_hbm, xs_hbm, dinv_hbm, zrows_hbm, zeros_hbm,
                zout_hbm, uout_hbm,
                sidx0, didx0, sidx1, didx1, val0, val1, rows0, rows1, zb_v,
                dinv_sp, z_sp, u_sp, isem0, isem1, gsem0, gsem1,
                vsem0, vsem1, usem0, usem1):
    cid = lax.axis_index("c")
    sid = lax.axis_index("s")
    w = cid * _NS + sid
    e0 = w * _EPW
    r0 = sid * _RPS
    pltpu.sync_copy(dinv_hbm.at[pl.ds(r0, _RPS)], dinv_sp.at[pl.ds(r0, _RPS)])
    pltpu.sync_copy(zrows_hbm, rows0)
    pltpu.sync_copy(zeros_hbm, zb_v)
    for r in range(_RPS // _K):
        pltpu.sync_copy(rows0, z_sp.at[pl.ds(r0 + r * _K, _K), :])
    pltpu.sync_copy(zb_v, u_sp.at[pl.ds(r0, _RPS)])
    plsc.subcore_barrier()

    def fetch_idx(c, sic, dic, isc):
        pltpu.async_copy(src_hbm.at[pl.ds(e0 + c * _K, _K)], sic, isc)
        pltpu.async_copy(dst_hbm.at[pl.ds(e0 + c * _K, _K)], dic, isc)

    def wait_idx(sic, dic, isc):
        pltpu.make_async_copy(src_hbm.at[pl.ds(0, _K)], sic, isc).wait()
        pltpu.make_async_copy(src_hbm.at[pl.ds(0, _K)], dic, isc).wait()

    fetch_idx(0, sidx0, didx0, isem0)
    wait_idx(sidx0, didx0, isem0)
    pltpu.async_copy(xs_hbm.at[sidx0], rows0, gsem0)
    pltpu.async_copy(dinv_sp.at[didx0], val0, vsem0)
    fetch_idx(1, sidx1, didx1, isem1)

    def process(c, sic, dic, rc, vc, gc, isc, vsc, usc,
                sin, din, rn, vn, gn, isn, vsn, usn,
                pre_gather, pre_idx):
        if pre_gather:
            wait_idx(sin, din, isn)
            pltpu.async_copy(xs_hbm.at[sin], rn, gn)
            pltpu.async_copy(dinv_sp.at[din], vn, vsn)
        pltpu.make_async_copy(dinv_sp.at[dic], vc, vsc).wait()
        pltpu.async_copy(vc, u_sp.at[sic], usc, add=True)
        pltpu.make_async_copy(xs_hbm.at[sic], rc, gc).wait()
        pltpu.sync_copy(rc, z_sp.at[dic], add=True)
        pltpu.make_async_copy(vc, u_sp.at[sic], usc).wait()
        if pre_idx:
            @pl.when(c + 2 < _NCHUNK)
            def _():
                fetch_idx(c + 2, sic, dic, isc)

    b0 = (sidx0, didx0, rows0, val0, gsem0, isem0, vsem0, usem0)
    b1 = (sidx1, didx1, rows1, val1, gsem1, isem1, vsem1, usem1)

    def step(p, carry):
        c = 2 * p
        process(c, *b0, *b1, True, True)
        process(c + 1, *b1, *b0, True, True)
        return carry

    lax.fori_loop(0, _NCHUNK // 2, step, 0)
    process(_NCHUNK - 1, *b0, *b1, False, False)
    plsc.subcore_barrier()
    pltpu.sync_copy(z_sp.at[pl.ds(r0, _RPS), :],
                    zout_hbm.at[cid, pl.ds(r0, _RPS), :])
    pltpu.sync_copy(u_sp.at[pl.ds(r0, _RPS)],
                    uout_hbm.at[cid, pl.ds(r0, _RPS)])


_BA = 2048


def _prep_body(degp_ref, x_ref, dinv_ref, xs_ref):
    deg = degp_ref[:, 0:1] + degp_ref[:, 1:2] + 1.0
    dv = lax.rsqrt(deg)
    dinv_ref[...] = dv
    xs_ref[...] = x_ref[...] * dv


def _tc_prep(degp_t, x_pad):
    return pl.pallas_call(
        _prep_body,
        grid=(_NPAD // _BA,),
        in_specs=[
            pl.BlockSpec((_BA, 2), lambda i: (i, 0)),
            pl.BlockSpec((_BA, _D), lambda i: (i, 0)),
        ],
        out_specs=[
            pl.BlockSpec((_BA, 1), lambda i: (i, 0)),
            pl.BlockSpec((_BA, _D), lambda i: (i, 0)),
        ],
        out_shape=[
            jax.ShapeDtypeStruct((_NPAD, 1), jnp.float32),
            jax.ShapeDtypeStruct((_NPAD, _D), jnp.float32),
        ],
    )(degp_t, x_pad)


_BB = 1024


def _red_body(zp_ref, up_ref, dinv_ref, x_ref, w1_ref, b1_ref, s_ref):
    i = pl.program_id(0)
    dv = dinv_ref[...]
    z = dv * (zp_ref[0] + zp_ref[1]) + dv * dv * x_ref[...]
    y = jnp.dot(z, w1_ref[...], preferred_element_type=jnp.float32) + b1_ref[...]
    y = jnp.where(y >= 0.0, y, 0.01 * y)
    u = up_ref[:, 0:1] + up_ref[:, 1:2]
    wv = dv * (u + dv)
    rows = i * _BB + lax.broadcasted_iota(jnp.int32, (_BB, 1), 0)
    wv = jnp.where(rows < _N, wv, 0.0)
    part = jnp.sum(wv * y, axis=0, keepdims=True)

    @pl.when(i == 0)
    def _():
        s_ref[...] = part

    @pl.when(i > 0)
    def _():
        s_ref[...] += part


def _tc_reduce(zp, up_t, dinv2, x_pad, W1, b1r):
    return pl.pallas_call(
        _red_body,
        grid=(_NPAD // _BB,),
        in_specs=[
            pl.BlockSpec((_NC, _BB, _D), lambda i: (0, i, 0)),
            pl.BlockSpec((_BB, 2), lambda i: (i, 0)),
            pl.BlockSpec((_BB, 1), lambda i: (i, 0)),
            pl.BlockSpec((_BB, _D), lambda i: (i, 0)),
            pl.BlockSpec((_D, _HID), lambda i: (0, 0)),
            pl.BlockSpec((1, _HID), lambda i: (0, 0)),
        ],
        out_specs=pl.BlockSpec((1, _HID), lambda i: (0, 0)),
        out_shape=jax.ShapeDtypeStruct((1, _HID), jnp.float32),
    )(zp, up_t, dinv2, x_pad, W1, b1r)


def _head_body(ci_ref, s_ref, w2_ref, b2_ref, fg1_ref, fgb1_ref, fg2_ref,
               fgb2_ref, eg1_ref, egb1_ref, eg2_ref, egb2_ref, tf_ref, p_ref):
    g = jnp.dot(s_ref[...], w2_ref[...],
                preferred_element_type=jnp.float32) * (1.0 / _N) + b2_ref[...]
    h1 = jnp.maximum(jnp.dot(g, fg1_ref[...],
                             preferred_element_type=jnp.float32) + fgb1_ref[...], 0.0)
    tf_ref[...] = jnp.dot(h1, fg2_ref[...],
                          preferred_element_type=jnp.float32) + fgb2_ref[...]
    h2 = jnp.maximum(jnp.dot(g, eg1_ref[...],
                             preferred_element_type=jnp.float32) + egb1_ref[...], 0.0)
    ew = jnp.dot(h2, eg2_ref[...],
                 preferred_element_type=jnp.float32) + egb2_ref[...]
    p = jax.nn.sigmoid(ew)
    k = lax.broadcasted_iota(jnp.int32, (1, _T * _T), 1)
    ii = k // _T
    jj = k - ii * _T
    ci = ci_ref[0]
    p = jnp.where((ii == ci) | (jj == ci), 1.0, p)
    p = jnp.where(ii == jj, 0.0, p)
    p_ref[...] = p


def _tc_head(ci_arr, s, W2, b2r, fgW1, fgb1r, fgW2, fgb2r, egW1, egb1r, egW2, egb2r):
    return pl.pallas_call(
        _head_body,
        in_specs=[pl.BlockSpec(memory_space=pltpu.SMEM)] + [pl.BlockSpec()] * 11,
        out_specs=[pl.BlockSpec(), pl.BlockSpec()],
        out_shape=[
            jax.ShapeDtypeStruct((1, _T * _D), jnp.float32),
            jax.ShapeDtypeStruct((1, _T * _T), jnp.float32),
        ],
    )(ci_arr, s, W2, b2r, fgW1, fgb1r, fgW2, fgb2r, egW1, egb1r, egW2, egb2r)


def kernel(graph_x, graph_edge_index, connected_trigger_node_index,
           target_node_index, W1, b1, W2, b2, fgW1, fgb1, fgW2, fgb2,
           egW1, egb1, egW2, egb2):
    ci = jnp.asarray(connected_trigger_node_index, dtype=jnp.int32)
    ti = jnp.asarray(target_node_index, dtype=jnp.int32)
    x_pad = jnp.pad(graph_x, ((0, _NPAD - _N), (0, 0)))
    zeros1d = jnp.zeros((_RPS,), jnp.float32)
    zrows = jnp.zeros((_K, _D), jnp.float32)

    src = graph_edge_index[0]
    dst = graph_edge_index[1]
    degp = _sc_deg(dst, zeros1d)
    dinv2, xs = _tc_prep(degp.T, x_pad)
    zp, up = _sc_scatter(src, dst, xs, dinv2.reshape(_NPAD),
                         zrows, zeros1d)
    s = _tc_reduce(zp, up.T, dinv2, x_pad, W1, b1.reshape(1, _HID))
    tf, p = _tc_head(ci.reshape(1), s, W2, b2.reshape(1, _HID),
                     fgW1, fgb1.reshape(1, _HID), fgW2, fgb2.reshape(1, _T * _D),
                     egW1, egb1.reshape(1, _HID), egW2, egb2.reshape(1, _T * _T))

    trigger_features = tf.reshape(_T, _D)
    trigger_edge_weight = p.reshape(_T * _T)[jnp.asarray(_OFF_FLAT)]
    rows = jnp.asarray(_OFF[:, 0], jnp.int32)
    cols = jnp.asarray(_OFF[:, 1], jnp.int32)
    trigger_edge_index = jnp.stack([rows + _N, cols + _N]).astype(jnp.int32)
    tte = jnp.stack([jnp.stack([ci + _N, ti]),
                     jnp.stack([ti, ci + _N])]).astype(jnp.int32)
    combined_x = jnp.concatenate([graph_x, trigger_features], axis=0)
    combined_edge_index = jnp.concatenate(
        [graph_edge_index, tte, trigger_edge_index], axis=1)
    return (combined_x, combined_edge_index, trigger_edge_weight)


# speedup vs baseline: 46.9845x; 1.0015x over previous
import functools

import numpy as np
import jax
import jax.numpy as jnp
from jax import lax
from jax.experimental import pallas as pl
from jax.experimental.pallas import tpu as pltpu
from jax.experimental.pallas import tpu_sc as plsc

_N = 10000
_E = 320000
_D = 128
_HID = 256
_T = 16

_NC, _NS, _L = 2, 16, 16
_NW = _NC * _NS
_NPAD = 10240
_RPS = _NPAD // _NS
_EPW = _E // _NW
_K = 80
_NCHUNK = _EPW // _K

_mesh = plsc.VectorSubcoreMesh(core_axis_name="c", subcore_axis_name="s")

_OFF = np.array([(i, j) for i in range(_T) for j in range(_T) if i != j],
                dtype=np.int32)
_OFF_FLAT = _OFF[:, 0] * _T + _OFF[:, 1]


@functools.partial(
    pl.kernel,
    out_type=jax.ShapeDtypeStruct((_NC, _NPAD), jnp.float32),
    mesh=_mesh,
    scratch_types=[
        pltpu.VMEM((_K,), jnp.int32),
        pltpu.VMEM((_K,), jnp.int32),
        pltpu.VMEM((_K,), jnp.int32),
        pltpu.VMEM((_K,), jnp.float32),
        pltpu.VMEM((_RPS,), jnp.float32),
        pltpu.VMEM_SHARED((_NPAD,), jnp.float32),
        pltpu.SemaphoreType.DMA,
        pltpu.SemaphoreType.DMA,
        pltpu.SemaphoreType.DMA,
        pltpu.SemaphoreType.DMA,
        pltpu.SemaphoreType.DMA,
        pltpu.SemaphoreType.DMA,
    ],
)
def _sc_deg(dst_hbm, zeros_hbm, out_hbm, didx0, didx1, didx2, ones_v, zb_v,
            hist_sp, isem0, isem1, isem2, ssem0, ssem1, ssem2):
    cid = lax.axis_index("c")
    sid = lax.axis_index("s")
    w = cid * _NS + sid
    e0 = w * _EPW
    pltpu.sync_copy(zeros_hbm, zb_v)
    pltpu.sync_copy(zb_v, hist_sp.at[pl.ds(sid * _RPS, _RPS)])
    for j in range(_K // _L):
        ones_v[pl.ds(j * _L, _L)] = jnp.ones((_L,), jnp.float32)
    plsc.subcore_barrier()
    bufs = ((didx0, isem0, ssem0), (didx1, isem1, ssem1), (didx2, isem2, ssem2))
    pltpu.async_copy(dst_hbm.at[pl.ds(e0, _K)], didx0, isem0)
    pltpu.async_copy(dst_hbm.at[pl.ds(e0 + _K, _K)], didx1, isem1)

    def process(c, cur, prv):
        dic, isc, ssc = cur
        dip, isp, ssp = prv
        pltpu.make_async_copy(dst_hbm.at[pl.ds(0, _K)], dic, isc).wait()
        pltpu.async_copy(ones_v, hist_sp.at[dic], ssc, add=True)

        @pl.when(c >= 1)
        def _():
            pltpu.make_async_copy(ones_v, hist_sp.at[dip], ssp).wait()

        @pl.when(c + 2 < _NCHUNK)
        def _():
            pltpu.async_copy(dst_hbm.at[pl.ds(e0 + (c + 2) * _K, _K)],
                             dip, isp)

    def step(p, carry):
        c = 3 * p
        process(c, bufs[0], bufs[2])
        process(c + 1, bufs[1], bufs[0])
        process(c + 2, bufs[2], bufs[1])
        return carry

    lax.fori_loop(0, _NCHUNK // 3, step, 0)
    process(_NCHUNK - 2, bufs[0], bufs[2])
    process(_NCHUNK - 1, bufs[1], bufs[0])
    pltpu.make_async_copy(ones_v, hist_sp.at[didx1], ssem1).wait()
    plsc.subcore_barrier()
    pltpu.sync_copy(hist_sp.at[pl.ds(sid * _RPS, _RPS)],
                    out_hbm.at[cid, pl.ds(sid * _RPS, _RPS)])


@functools.partial(
    pl.kernel,
    out_type=(
        jax.ShapeDtypeStruct((_NC, _NPAD, _D), jnp.float32),
        jax.ShapeDtypeStruct((_NC, _NPAD), jnp.float32),
    ),
    mesh=_mesh,
    scratch_types=[
        pltpu.VMEM((_K,), jnp.int32),
        pltpu.VMEM((_K,), jnp.int32),
        pltpu.VMEM((_K,), jnp.int32),
        pltpu.VMEM((_K,), jnp.int32),
        pltpu.VMEM((_K,), jnp.float32),
        pltpu.VMEM((_K,), jnp.float32),
        pltpu.VMEM((_K, _D), jnp.float32),
        pltpu.VMEM((_K, _D), jnp.float32),
        pltpu.VMEM((_RPS,), jnp.float32),
        pltpu.VMEM_SHARED((_NPAD,), jnp.float32),
        pltpu.VMEM_SHARED((_NPAD, _D), jnp.float32),
        pltpu.VMEM_SHARED((_NPAD,), jnp.float32),
        pltpu.SemaphoreType.DMA,
        pltpu.SemaphoreType.DMA,
        pltpu.SemaphoreType.DMA,
        pltpu.SemaphoreType.DMA,
        pltpu.SemaphoreType.DMA,
        pltpu.SemaphoreType.DMA,
        pltpu.SemaphoreType.DMA,
        pltpu.SemaphoreType.DMA,
    ],
)
def _sc_scatter(src_hbm, dst_hbm, xs_hbm, dinv_hbm, zrows_hbm, zeros_hbm,
                zout_hbm, uout_hbm,
                sidx0, didx0, sidx1, didx1, val0, val1, rows0, rows1, zb_v,
                dinv_sp, z_sp, u_sp, isem0, isem1, gsem0, gsem1,
                vsem0, vsem1, usem0, usem1):
    cid = lax.axis_index("c")
    sid = lax.axis_index("s")
    w = cid * _NS + sid
    e0 = w * _EPW
    r0 = sid * _RPS
    pltpu.sync_copy(dinv_hbm.at[pl.ds(r0, _RPS)], dinv_sp.at[pl.ds(r0, _RPS)])
    pltpu.sync_copy(zrows_hbm, rows0)
    pltpu.sync_copy(zeros_hbm, zb_v)
    for r in range(_RPS // _K):
        pltpu.sync_copy(rows0, z_sp.at[pl.ds(r0 + r * _K, _K), :])
    pltpu.sync_copy(zb_v, u_sp.at[pl.ds(r0, _RPS)])
    plsc.subcore_barrier()

    def fetch_idx(c, sic, dic, isc):
        pltpu.async_copy(src_hbm.at[pl.ds(e0 + c * _K, _K)], sic, isc)
        pltpu.async_copy(dst_hbm.at[pl.ds(e0 + c * _K, _K)], dic, isc)

    def wait_idx(sic, dic, isc):
        pltpu.make_async_copy(src_hbm.at[pl.ds(0, _K)], sic, isc).wait()
        pltpu.make_async_copy(src_hbm.at[pl.ds(0, _K)], dic, isc).wait()

    def gather_rows(sic, rc, gc):
        pltpu.async_copy(xs_hbm.at[sic.at[pl.ds(0, _K // 2)]],
                         rc.at[pl.ds(0, _K // 2), :], gc)
        pltpu.async_copy(xs_hbm.at[sic.at[pl.ds(_K // 2, _K // 2)]],
                         rc.at[pl.ds(_K // 2, _K // 2), :], gc)

    def wait_rows(sic, rc, gc):
        pltpu.make_async_copy(xs_hbm.at[sic.at[pl.ds(0, _K // 2)]],
                              rc.at[pl.ds(0, _K // 2), :], gc).wait()
        pltpu.make_async_copy(xs_hbm.at[sic.at[pl.ds(_K // 2, _K // 2)]],
                              rc.at[pl.ds(_K // 2, _K // 2), :], gc).wait()

    fetch_idx(0, sidx0, didx0, isem0)
    wait_idx(sidx0, didx0, isem0)
    gather_rows(sidx0, rows0, gsem0)
    pltpu.async_copy(dinv_sp.at[didx0], val0, vsem0)
    fetch_idx(1, sidx1, didx1, isem1)

    def process(c, sic, dic, rc, vc, gc, isc, vsc, usc,
                sin, din, rn, vn, gn, isn, vsn, usn,
                pre_gather, pre_idx):
        if pre_gather:
            wait_idx(sin, din, isn)
            gather_rows(sin, rn, gn)
            pltpu.async_copy(dinv_sp.at[din], vn, vsn)
        pltpu.make_async_copy(dinv_sp.at[dic], vc, vsc).wait()
        pltpu.async_copy(vc, u_sp.at[sic], usc, add=True)
        wait_rows(sic, rc, gc)
        pltpu.sync_copy(rc, z_sp.at[dic], add=True)
        pltpu.make_async_copy(vc, u_sp.at[sic], usc).wait()
        if pre_idx:
            @pl.when(c + 2 < _NCHUNK)
            def _():
                fetch_idx(c + 2, sic, dic, isc)

    b0 = (sidx0, didx0, rows0, val0, gsem0, isem0, vsem0, usem0)
    b1 = (sidx1, didx1, rows1, val1, gsem1, isem1, vsem1, usem1)

    def step(p, carry):
        c = 2 * p
        process(c, *b0, *b1, True, True)
        process(c + 1, *b1, *b0, True, True)
        return carry

    lax.fori_loop(0, _NCHUNK // 2, step, 0)
    process(_NCHUNK - 1, *b0, *b1, False, False)
    plsc.subcore_barrier()
    pltpu.sync_copy(z_sp.at[pl.ds(r0, _RPS), :],
                    zout_hbm.at[cid, pl.ds(r0, _RPS), :])
    pltpu.sync_copy(u_sp.at[pl.ds(r0, _RPS)],
                    uout_hbm.at[cid, pl.ds(r0, _RPS)])


_BA = 2048


def _prep_body(degp_ref, x_ref, dinv_ref, xs_ref):
    deg = degp_ref[:, 0:1] + degp_ref[:, 1:2] + 1.0
    dv = lax.rsqrt(deg)
    dinv_ref[...] = dv
    xs_ref[...] = x_ref[...] * dv


def _tc_prep(degp_t, x_pad):
    return pl.pallas_call(
        _prep_body,
        grid=(_NPAD // _BA,),
        in_specs=[
            pl.BlockSpec((_BA, 2), lambda i: (i, 0)),
            pl.BlockSpec((_BA, _D), lambda i: (i, 0)),
        ],
        out_specs=[
            pl.BlockSpec((_BA, 1), lambda i: (i, 0)),
            pl.BlockSpec((_BA, _D), lambda i: (i, 0)),
        ],
        out_shape=[
            jax.ShapeDtypeStruct((_NPAD, 1), jnp.float32),
            jax.ShapeDtypeStruct((_NPAD, _D), jnp.float32),
        ],
    )(degp_t, x_pad)


_BB = 1024


def _red_body(zp_ref, up_ref, dinv_ref, x_ref, w1_ref, b1_ref, s_ref):
    i = pl.program_id(0)
    dv = dinv_ref[...]
    z = dv * (zp_ref[0] + zp_ref[1]) + dv * dv * x_ref[...]
    y = jnp.dot(z, w1_ref[...], preferred_element_type=jnp.float32) + b1_ref[...]
    y = jnp.where(y >= 0.0, y, 0.01 * y)
    u = up_ref[:, 0:1] + up_ref[:, 1:2]
    wv = dv * (u + dv)
    rows = i * _BB + lax.broadcasted_iota(jnp.int32, (_BB, 1), 0)
    wv = jnp.where(rows < _N, wv, 0.0)
    part = jnp.sum(wv * y, axis=0, keepdims=True)

    @pl.when(i == 0)
    def _():
        s_ref[...] = part

    @pl.when(i > 0)
    def _():
        s_ref[...] += part


def _tc_reduce(zp, up_t, dinv2, x_pad, W1, b1r):
    return pl.pallas_call(
        _red_body,
        grid=(_NPAD // _BB,),
        in_specs=[
            pl.BlockSpec((_NC, _BB, _D), lambda i: (0, i, 0)),
            pl.BlockSpec((_BB, 2), lambda i: (i, 0)),
            pl.BlockSpec((_BB, 1), lambda i: (i, 0)),
            pl.BlockSpec((_BB, _D), lambda i: (i, 0)),
            pl.BlockSpec((_D, _HID), lambda i: (0, 0)),
            pl.BlockSpec((1, _HID), lambda i: (0, 0)),
        ],
        out_specs=pl.BlockSpec((1, _HID), lambda i: (0, 0)),
        out_shape=jax.ShapeDtypeStruct((1, _HID), jnp.float32),
    )(zp, up_t, dinv2, x_pad, W1, b1r)


def _head_body(ci_ref, s_ref, w2_ref, b2_ref, fg1_ref, fgb1_ref, fg2_ref,
               fgb2_ref, eg1_ref, egb1_ref, eg2_ref, egb2_ref, tf_ref, p_ref):
    g = jnp.dot(s_ref[...], w2_ref[...],
                preferred_element_type=jnp.float32) * (1.0 / _N) + b2_ref[...]
    h1 = jnp.maximum(jnp.dot(g, fg1_ref[...],
                             preferred_element_type=jnp.float32) + fgb1_ref[...], 0.0)
    tf_ref[...] = jnp.dot(h1, fg2_ref[...],
                          preferred_element_type=jnp.float32) + fgb2_ref[...]
    h2 = jnp.maximum(jnp.dot(g, eg1_ref[...],
                             preferred_element_type=jnp.float32) + egb1_ref[...], 0.0)
    ew = jnp.dot(h2, eg2_ref[...],
                 preferred_element_type=jnp.float32) + egb2_ref[...]
    p = jax.nn.sigmoid(ew)
    k = lax.broadcasted_iota(jnp.int32, (1, _T * _T), 1)
    ii = k // _T
    jj = k - ii * _T
    ci = ci_ref[0]
    p = jnp.where((ii == ci) | (jj == ci), 1.0, p)
    p = jnp.where(ii == jj, 0.0, p)
    p_ref[...] = p


def _tc_head(ci_arr, s, W2, b2r, fgW1, fgb1r, fgW2, fgb2r, egW1, egb1r, egW2, egb2r):
    return pl.pallas_call(
        _head_body,
        in_specs=[pl.BlockSpec(memory_space=pltpu.SMEM)] + [pl.BlockSpec()] * 11,
        out_specs=[pl.BlockSpec(), pl.BlockSpec()],
        out_shape=[
            jax.ShapeDtypeStruct((1, _T * _D), jnp.float32),
            jax.ShapeDtypeStruct((1, _T * _T), jnp.float32),
        ],
    )(ci_arr, s, W2, b2r, fgW1, fgb1r, fgW2, fgb2r, egW1, egb1r, egW2, egb2r)


def kernel(graph_x, graph_edge_index, connected_trigger_node_index,
           target_node_index, W1, b1, W2, b2, fgW1, fgb1, fgW2, fgb2,
           egW1, egb1, egW2, egb2):
    ci = jnp.asarray(connected_trigger_node_index, dtype=jnp.int32)
    ti = jnp.asarray(target_node_index, dtype=jnp.int32)
    x_pad = jnp.pad(graph_x, ((0, _NPAD - _N), (0, 0)))
    zeros1d = jnp.zeros((_RPS,), jnp.float32)
    zrows = jnp.zeros((_K, _D), jnp.float32)

    src = graph_edge_index[0]
    dst = graph_edge_index[1]
    degp = _sc_deg(dst, zeros1d)
    dinv2, xs = _tc_prep(degp.T, x_pad)
    zp, up = _sc_scatter(src, dst, xs, dinv2.reshape(_NPAD),
                         zrows, zeros1d)
    s = _tc_reduce(zp, up.T, dinv2, x_pad, W1, b1.reshape(1, _HID))
    tf, p = _tc_head(ci.reshape(1), s, W2, b2.reshape(1, _HID),
                     fgW1, fgb1.reshape(1, _HID), fgW2, fgb2.reshape(1, _T * _D),
                     egW1, egb1.reshape(1, _HID), egW2, egb2.reshape(1, _T * _T))

    trigger_features = tf.reshape(_T, _D)
    trigger_edge_weight = p.reshape(_T * _T)[jnp.asarray(_OFF_FLAT)]
    rows = jnp.asarray(_OFF[:, 0], jnp.int32)
    cols = jnp.asarray(_OFF[:, 1], jnp.int32)
    trigger_edge_index = jnp.stack([rows + _N, cols + _N]).astype(jnp.int32)
    tte = jnp.stack([jnp.stack([ci + _N, ti]),
                     jnp.stack([ti, ci + _N])]).astype(jnp.int32)
    combined_x = jnp.concatenate([graph_x, trigger_features], axis=0)
    combined_edge_index = jnp.concatenate(
        [graph_edge_index, tte, trigger_edge_index], axis=1)
    return (combined_x, combined_edge_index, trigger_edge_weight)
